# baseline (speedup 1.0000x reference)
import functools

import jax
import jax.numpy as jnp
import numpy as np
from jax import lax
from jax.experimental import pallas as pl
from jax.experimental.pallas import tpu as pltpu
from jax.experimental.pallas import tpu_sc as plsc

B = 4096
D = 64
F = 100000
NCAT = 26
NNUM = 13
S = 40
WPAD = 16
NCORE = 2
NSUB = 16
NW = NCORE * NSUB
RW = B // NW
NBUF = 4
LANE = 16
CB = RW // LANE

_PERM = np.concatenate([
    np.arange(0, 32, 2), np.arange(1, 32, 2),
    np.arange(32, 64, 2), np.arange(33, 64, 2)])


def _sc_pool(idx2d, w2d, emb_bf, lin_flat):
    mesh = plsc.VectorSubcoreMesh(
        core_axis_name="c", subcore_axis_name="s",
        num_cores=NCORE, num_subcores=NSUB)

    @functools.partial(
        pl.kernel,
        out_type=(jax.ShapeDtypeStruct((B, D), jnp.float32),
                  jax.ShapeDtypeStruct((B,), jnp.float32)),
        mesh=mesh,
        scratch_types=[
            pltpu.VMEM((RW, S), jnp.int32),
            pltpu.VMEM((RW, WPAD), jnp.float32),
            pltpu.VMEM((F,), jnp.float32),
            pltpu.VMEM((NBUF, S, D), jnp.float32),
            pltpu.VMEM((RW, D), jnp.float32),
            pltpu.VMEM((RW,), jnp.float32),
            pltpu.SemaphoreType.DMA,
        ] + [pltpu.SemaphoreType.DMA for _ in range(NBUF)],
        compiler_params=pltpu.CompilerParams(
            use_tc_tiling_on_sc=False, needs_layout_passes=False),
    )
    def k(idx_h, w_h, emb_h, lin_h, so_h, fo_h,
          idx_v, w_v, lin_t, ebuf, so_v, fo_v, lsem, *esems):
        wid = lax.axis_index("s") * NCORE + lax.axis_index("c")
        base = wid * RW

        pltpu.async_copy(lin_h, lin_t, lsem)

        pltpu.sync_copy(idx_h.at[pl.ds(base, RW)], idx_v)
        pltpu.sync_copy(w_h.at[pl.ds(base, RW)], w_v)

        for g in range(NBUF):
            pltpu.async_copy(
                emb_h.at[idx_v.at[g]], ebuf.at[g], esems[g])

        zi = jnp.zeros((LANE,), jnp.int32)
        rows16 = lax.iota(jnp.int32, LANE)

        pltpu.make_async_copy(lin_h, lin_t, lsem).wait()

        for gq in range(CB):
            r16 = rows16 + gq * LANE

            def cate_acc(j, acc):
                idx16 = plsc.load_gather(idx_v, [r16, zi + j])
                return acc + plsc.load_gather(lin_t, [idx16])
            acc = lax.fori_loop(0, NCAT, cate_acc,
                                jnp.zeros((LANE,), jnp.float32))
            for t in range(NNUM):
                idx16 = plsc.load_gather(idx_v, [r16, zi + (NCAT + t)])
                lin16 = plsc.load_gather(lin_t, [idx16])
                wv = plsc.load_gather(w_v, [r16, zi + t])
                acc = acc + lin16 * wv
            fo_v[pl.ds(gq * LANE, LANE)] = acc

        def ring_body(o, carry):
            for slot in range(NBUF):
                i = o * NBUF + slot
                pltpu.make_async_copy(
                    emb_h.at[idx_v.at[i]], ebuf.at[slot], esems[slot]).wait()
                s = [jnp.zeros((LANE,), jnp.float32) for _ in range(4)]
                ss = [jnp.zeros((LANE,), jnp.float32) for _ in range(4)]
                for j in range(NCAT):
                    for c in range(4):
                        v = ebuf[slot, j, pl.ds(c * LANE, LANE)]
                        s[c] = s[c] + v
                        ss[c] = ss[c] + v * v
                for t in range(NNUM):
                    wb = plsc.load_gather(w_v, [zi + i, zi + t])
                    for c in range(4):
                        v = ebuf[slot, NCAT + t, pl.ds(c * LANE, LANE)] * wb
                        s[c] = s[c] + v
                        ss[c] = ss[c] + v * v
                for c in range(4):
                    so_v[i, pl.ds(c * LANE, LANE)] = (
                        0.5 * (s[c] * s[c] - ss[c]))

                @pl.when(i + NBUF < RW)
                def _():
                    pltpu.async_copy(
                        emb_h.at[idx_v.at[i + NBUF]], ebuf.at[slot],
                        esems[slot])
            return carry
        lax.fori_loop(0, RW // NBUF, ring_body, 0)

        pltpu.sync_copy(so_v, so_h.at[pl.ds(base, RW)])
        pltpu.sync_copy(fo_v, fo_h.at[pl.ds(base, RW)])

    return k(idx2d, w2d, emb_bf, lin_flat)


def _mlp(so, fo, W1, b1, W2, b2, W3t, b3):
    GB = 4
    BB = B // GB

    def body(so_ref, fo_ref, w1_ref, b1_ref, w2_ref, b2_ref, w3_ref, b3_ref,
             out_ref):
        h = jnp.dot(so_ref[...], w1_ref[...],
                    preferred_element_type=jnp.float32)
        h = jnp.maximum(h + b1_ref[...], 0.0)
        h = jnp.dot(h, w2_ref[...], preferred_element_type=jnp.float32)
        h = jnp.maximum(h + b2_ref[...], 0.0)
        z = jnp.sum(h * w3_ref[...], axis=1, keepdims=True) + b3_ref[0, 0]
        out_ref[...] = fo_ref[...] + jax.nn.sigmoid(z)

    return pl.pallas_call(
        body,
        grid=(GB,),
        in_specs=[
            pl.BlockSpec((BB, D), lambda i: (i, 0)),
            pl.BlockSpec((BB, 1), lambda i: (i, 0)),
            pl.BlockSpec((D, 64), lambda i: (0, 0)),
            pl.BlockSpec((1, 64), lambda i: (0, 0)),
            pl.BlockSpec((64, 32), lambda i: (0, 0)),
            pl.BlockSpec((1, 32), lambda i: (0, 0)),
            pl.BlockSpec((1, 32), lambda i: (0, 0)),
            pl.BlockSpec((1, 1), lambda i: (0, 0)),
        ],
        out_specs=pl.BlockSpec((BB, 1), lambda i: (i, 0)),
        out_shape=jax.ShapeDtypeStruct((B, 1), jnp.float32),
    )(so, fo, W1, b1, W2, b2, W3t, b3)


def kernel(category_index, numerical_index, numerical_value, emb_table,
           lin_table, W1, b1, W2, b2, W3, b3):
    ci = category_index.astype(jnp.int32)
    ni = numerical_index.astype(jnp.int32)
    nv = numerical_value.astype(jnp.float32)
    idx2d = jnp.concatenate([ci, ni, jnp.zeros((B, 1), jnp.int32)], axis=1)
    w2d = jnp.concatenate(
        [nv, jnp.zeros((B, WPAD - NNUM), jnp.float32)], axis=1)
    emb_bf = emb_table
    lin_flat = lin_table[:, 0]

    so, fo = _sc_pool(idx2d, w2d, emb_bf, lin_flat)
    out = _mlp(so, fo[:, None], W1, b1.reshape(1, 64),
               W2, b2.reshape(1, 32), W3.T, b3.reshape(1, 1))
    return out


# speedup vs baseline: 1.0527x; 1.0527x over previous
import functools

import jax
import jax.numpy as jnp
import numpy as np
from jax import lax
from jax.experimental import pallas as pl
from jax.experimental.pallas import tpu as pltpu
from jax.experimental.pallas import tpu_sc as plsc

B = 4096
D = 64
F = 100000
NCAT = 26
NNUM = 13
S = 40
WPAD = 16
NCORE = 2
NSUB = 16
NW = NCORE * NSUB
RW = B // NW
NBUF = 8
LANE = 16
CB = RW // LANE

_PERM = np.concatenate([
    np.arange(0, 32, 2), np.arange(1, 32, 2),
    np.arange(32, 64, 2), np.arange(33, 64, 2)])


def _sc_pool(idx2d, w2d, emb_bf, lin_flat):
    mesh = plsc.VectorSubcoreMesh(
        core_axis_name="c", subcore_axis_name="s",
        num_cores=NCORE, num_subcores=NSUB)

    @functools.partial(
        pl.kernel,
        out_type=(jax.ShapeDtypeStruct((B, D), jnp.float32),
                  jax.ShapeDtypeStruct((B,), jnp.float32)),
        mesh=mesh,
        scratch_types=[
            pltpu.VMEM((RW, S), jnp.int32),
            pltpu.VMEM((RW, WPAD), jnp.float32),
            pltpu.VMEM((F,), jnp.float32),
            pltpu.VMEM((NBUF, S, D), jnp.bfloat16),
            pltpu.VMEM((RW, D), jnp.float32),
            pltpu.VMEM((RW,), jnp.float32),
            pltpu.SemaphoreType.DMA,
        ] + [pltpu.SemaphoreType.DMA for _ in range(NBUF)],
        compiler_params=pltpu.CompilerParams(
            use_tc_tiling_on_sc=False, needs_layout_passes=False),
    )
    def k(idx_h, w_h, emb_h, lin_h, so_h, fo_h,
          idx_v, w_v, lin_t, ebuf, so_v, fo_v, lsem, *esems):
        wid = lax.axis_index("s") * NCORE + lax.axis_index("c")
        base = wid * RW

        pltpu.async_copy(lin_h, lin_t, lsem)

        pltpu.sync_copy(idx_h.at[pl.ds(base, RW)], idx_v)
        pltpu.sync_copy(w_h.at[pl.ds(base, RW)], w_v)

        for g in range(NBUF):
            pltpu.async_copy(
                emb_h.at[idx_v.at[g]], ebuf.at[g], esems[g])

        zi = jnp.zeros((LANE,), jnp.int32)
        rows16 = lax.iota(jnp.int32, LANE)

        pltpu.make_async_copy(lin_h, lin_t, lsem).wait()

        for gq in range(CB):
            r16 = rows16 + gq * LANE

            def cate_acc(j, acc):
                idx16 = plsc.load_gather(idx_v, [r16, zi + j])
                return acc + plsc.load_gather(lin_t, [idx16])
            acc = lax.fori_loop(0, NCAT, cate_acc,
                                jnp.zeros((LANE,), jnp.float32))
            for t in range(NNUM):
                idx16 = plsc.load_gather(idx_v, [r16, zi + (NCAT + t)])
                lin16 = plsc.load_gather(lin_t, [idx16])
                wv = plsc.load_gather(w_v, [r16, zi + t])
                acc = acc + lin16 * wv
            fo_v[pl.ds(gq * LANE, LANE)] = acc

        def ring_body(o, carry):
            for slot in range(NBUF):
                i = o * NBUF + slot
                pltpu.make_async_copy(
                    emb_h.at[idx_v.at[i]], ebuf.at[slot], esems[slot]).wait()
                s = [jnp.zeros((LANE,), jnp.float32) for _ in range(4)]
                ss = [jnp.zeros((LANE,), jnp.float32) for _ in range(4)]
                for j in range(NCAT):
                    for h in range(2):
                        pair = ebuf[slot, j, pl.ds(h * 32, 32)]
                        va, vb = plsc.unpack(
                            pair, format=plsc.PackFormat.INTERLEAVED,
                            preferred_element_type=jnp.float32)
                        s[2 * h] = s[2 * h] + va
                        ss[2 * h] = ss[2 * h] + va * va
                        s[2 * h + 1] = s[2 * h + 1] + vb
                        ss[2 * h + 1] = ss[2 * h + 1] + vb * vb
                for t in range(NNUM):
                    wb = plsc.load_gather(w_v, [zi + i, zi + t])
                    for h in range(2):
                        pair = ebuf[slot, NCAT + t, pl.ds(h * 32, 32)]
                        va, vb = plsc.unpack(
                            pair, format=plsc.PackFormat.INTERLEAVED,
                            preferred_element_type=jnp.float32)
                        va = va * wb
                        vb = vb * wb
                        s[2 * h] = s[2 * h] + va
                        ss[2 * h] = ss[2 * h] + va * va
                        s[2 * h + 1] = s[2 * h + 1] + vb
                        ss[2 * h + 1] = ss[2 * h + 1] + vb * vb
                for c in range(4):
                    so_v[i, pl.ds(c * LANE, LANE)] = (
                        0.5 * (s[c] * s[c] - ss[c]))

                @pl.when(i + NBUF < RW)
                def _():
                    pltpu.async_copy(
                        emb_h.at[idx_v.at[i + NBUF]], ebuf.at[slot],
                        esems[slot])
            return carry
        lax.fori_loop(0, RW // NBUF, ring_body, 0)

        pltpu.sync_copy(so_v, so_h.at[pl.ds(base, RW)])
        pltpu.sync_copy(fo_v, fo_h.at[pl.ds(base, RW)])

    return k(idx2d, w2d, emb_bf, lin_flat)


def _mlp(so, fo, W1, b1, W2, b2, W3t, b3):
    GB = 4
    BB = B // GB

    def body(so_ref, fo_ref, w1_ref, b1_ref, w2_ref, b2_ref, w3_ref, b3_ref,
             out_ref):
        h = jnp.dot(so_ref[...], w1_ref[...],
                    preferred_element_type=jnp.float32)
        h = jnp.maximum(h + b1_ref[...], 0.0)
        h = jnp.dot(h, w2_ref[...], preferred_element_type=jnp.float32)
        h = jnp.maximum(h + b2_ref[...], 0.0)
        z = jnp.sum(h * w3_ref[...], axis=1, keepdims=True) + b3_ref[0, 0]
        out_ref[...] = fo_ref[...] + jax.nn.sigmoid(z)

    return pl.pallas_call(
        body,
        grid=(GB,),
        in_specs=[
            pl.BlockSpec((BB, D), lambda i: (i, 0)),
            pl.BlockSpec((BB, 1), lambda i: (i, 0)),
            pl.BlockSpec((D, 64), lambda i: (0, 0)),
            pl.BlockSpec((1, 64), lambda i: (0, 0)),
            pl.BlockSpec((64, 32), lambda i: (0, 0)),
            pl.BlockSpec((1, 32), lambda i: (0, 0)),
            pl.BlockSpec((1, 32), lambda i: (0, 0)),
            pl.BlockSpec((1, 1), lambda i: (0, 0)),
        ],
        out_specs=pl.BlockSpec((BB, 1), lambda i: (i, 0)),
        out_shape=jax.ShapeDtypeStruct((B, 1), jnp.float32),
    )(so, fo, W1, b1, W2, b2, W3t, b3)


def kernel(category_index, numerical_index, numerical_value, emb_table,
           lin_table, W1, b1, W2, b2, W3, b3):
    ci = category_index.astype(jnp.int32)
    ni = numerical_index.astype(jnp.int32)
    nv = numerical_value.astype(jnp.float32)
    idx2d = jnp.concatenate([ci, ni, jnp.zeros((B, 1), jnp.int32)], axis=1)
    w2d = jnp.concatenate(
        [nv, jnp.zeros((B, WPAD - NNUM), jnp.float32)], axis=1)
    emb_bf = emb_table.astype(jnp.bfloat16)
    lin_flat = lin_table[:, 0]

    so, fo = _sc_pool(idx2d, w2d, emb_bf, lin_flat)
    out = _mlp(so, fo[:, None], W1[_PERM], b1.reshape(1, 64),
               W2, b2.reshape(1, 32), W3.T, b3.reshape(1, 1))
    return out
